# baseline (speedup 1.0000x reference)
import jax
import jax.numpy as jnp
from jax import lax
from jax.experimental import pallas as pl
from jax.experimental.pallas import tpu as pltpu
from jax.experimental.pallas import tpu_sc as plsc

M = 100000
D = 256
B = 16384
C = 100
CP = 128

NW = 32
RNG = 3128
CAP = 1024
PIECE = 128
NPIECE = CAP // PIECE
SRCN = 3136


def _copy_body(bx_r, lg_r, by_r, en_r,
               obx_r, olg_r, oby_r, oen_r):
    obx_r[...] = bx_r[...]
    olg_r[:, :C] = lg_r[...]
    oby_r[...] = by_r[...]
    oen_r[...] = en_r[...]


def _tc_copy(bx, logits_buf, by2, en2):
    grid = 100
    bm = M // grid
    return pl.pallas_call(
        _copy_body,
        grid=(grid,),
        in_specs=[
            pl.BlockSpec((bm, D), lambda i: (i, 0)),
            pl.BlockSpec((bm, C), lambda i: (i, 0)),
            pl.BlockSpec((8, 125), lambda i: (i, 0)),
            pl.BlockSpec((8, 125), lambda i: (i, 0)),
        ],
        out_specs=[
            pl.BlockSpec((bm, D), lambda i: (i, 0)),
            pl.BlockSpec((bm, CP), lambda i: (i, 0)),
            pl.BlockSpec((8, 125), lambda i: (i, 0)),
            pl.BlockSpec((8, 125), lambda i: (i, 0)),
        ],
        out_shape=[
            jax.ShapeDtypeStruct((M, D), jnp.float32),
            jax.ShapeDtypeStruct((M, CP), jnp.float32),
            jax.ShapeDtypeStruct((800, 125), jnp.int32),
            jax.ShapeDtypeStruct((800, 125), jnp.float32),
        ],
    )(bx, logits_buf, by2, en2)


def _padx_body(xlg_r, oxlg_r):
    oxlg_r[:, :C] = xlg_r[...]


def _tc_padx(xlg):
    grid = 128
    bb = B // grid
    return pl.pallas_call(
        _padx_body,
        grid=(grid,),
        in_specs=[pl.BlockSpec((bb, C), lambda i: (i, 0))],
        out_specs=pl.BlockSpec((bb, CP), lambda i: (i, 0)),
        out_shape=jax.ShapeDtypeStruct((B, CP), jnp.float32),
    )(xlg)


def _unpad_body(p_r, o_r):
    o_r[...] = p_r[:, :C]


def _tc_unpad(p):
    grid = 100
    bm = M // grid
    return pl.pallas_call(
        _unpad_body,
        grid=(grid,),
        in_specs=[pl.BlockSpec((bm, CP), lambda i: (i, 0))],
        out_specs=pl.BlockSpec((bm, C), lambda i: (i, 0)),
        out_shape=jax.ShapeDtypeStruct((M, C), jnp.float32),
    )(p)


def _sc_body(x_hbm, y_hbm, e_hbm, lg_hbm, idx_hbm,
             rbx, rby, ren, rlg,
             idx_v, src_v, rowl, jl, yv, ev, xbuf, lbuf, semg, sems):
    w = lax.axis_index("s") * 2 + lax.axis_index("c")
    base = jnp.where(w == NW - 1, M - RNG, w * RNG).astype(jnp.int32)
    lane = lax.iota(jnp.int32, 16)

    pltpu.sync_copy(idx_hbm, idx_v)

    def init_body(g, _):
        src_v[pl.ds(g * 16, 16)] = jnp.full((16,), -1, jnp.int32)
        return 0
    lax.fori_loop(0, SRCN // 16, init_body, 0)

    def scan_body(g, _):
        iv = idx_v[pl.ds(g * 16, 16)]
        loc = iv - base
        inr = (loc >= 0) & (loc < RNG)

        @pl.when(jnp.any(inr))
        def _():
            j = g * 16 + lane
            _, lastm = plsc.scan_count(loc, mask=inr)
            plsc.store_scatter(src_v, [loc], j, mask=lastm & inr)
        return 0
    lax.fori_loop(0, B // 16, scan_body, 0)

    def comp_body(g, carry):
        k, pmax = carry
        s = src_v[pl.ds(g * 16, 16)]
        m = s >= 0
        ones = jnp.where(m, 1, 0).astype(jnp.int32)
        pos = k + plsc.cumsum(ones) - 1
        m = m & (pos < CAP)
        rows = base + g * 16 + lane
        plsc.store_scatter(rowl, [pos >> 7, pos & 127], rows, mask=m)
        plsc.store_scatter(jl, [pos >> 7, pos & 127], s, mask=m)
        pair = jnp.where(m, rows * 16384 + s, -1)
        return k + jnp.sum(ones), jnp.maximum(pmax, jnp.max(pair))
    k, pmax = lax.fori_loop(0, SRCN // 16, comp_body,
                            (jnp.int32(0), jnp.int32(-1)))

    @pl.when(k > 0)
    def _():
        pad_row = pmax >> 14
        pad_j = pmax & 16383

        def pad_body(g, _):
            posv = g * 16 + lane
            m = posv >= k
            plsc.store_scatter(rowl, [posv >> 7, posv & 127],
                               jnp.full((16,), 1, jnp.int32) * pad_row, mask=m)
            plsc.store_scatter(jl, [posv >> 7, posv & 127],
                               jnp.full((16,), 1, jnp.int32) * pad_j, mask=m)
            return 0
        lax.fori_loop(k >> 4, CAP // 16, pad_body, 0)

        def upd_body(p, _):
            pltpu.make_async_copy(x_hbm.at[jl.at[p]], xbuf, semg).start()
            pltpu.make_async_copy(lg_hbm.at[jl.at[p]], lbuf, semg).start()
            pltpu.make_async_copy(y_hbm.at[jl.at[p]], yv, semg).start()
            pltpu.make_async_copy(e_hbm.at[jl.at[p]], ev, semg).start()
            pltpu.make_async_copy(x_hbm.at[jl.at[p]], xbuf, semg).wait()
            pltpu.make_async_copy(lg_hbm.at[jl.at[p]], lbuf, semg).wait()
            pltpu.make_async_copy(y_hbm.at[jl.at[p]], yv, semg).wait()
            pltpu.make_async_copy(e_hbm.at[jl.at[p]], ev, semg).wait()
            pltpu.make_async_copy(xbuf, rbx.at[rowl.at[p]], sems).start()
            pltpu.make_async_copy(lbuf, rlg.at[rowl.at[p]], sems).start()
            pltpu.make_async_copy(yv, rby.at[rowl.at[p]], sems).start()
            pltpu.make_async_copy(ev, ren.at[rowl.at[p]], sems).start()
            pltpu.make_async_copy(xbuf, rbx.at[rowl.at[p]], sems).wait()
            pltpu.make_async_copy(lbuf, rlg.at[rowl.at[p]], sems).wait()
            pltpu.make_async_copy(yv, rby.at[rowl.at[p]], sems).wait()
            pltpu.make_async_copy(ev, ren.at[rowl.at[p]], sems).wait()
            return 0
        lax.fori_loop(0, (k + PIECE - 1) >> 7, upd_body, 0)


_sc_update = pl.kernel(
    _sc_body,
    out_type=(),
    mesh=plsc.VectorSubcoreMesh(core_axis_name="c", subcore_axis_name="s"),
    compiler_params=pltpu.CompilerParams(needs_layout_passes=False,
                                         use_tc_tiling_on_sc=False),
    scratch_types=[
        pltpu.VMEM((B,), jnp.int32),
        pltpu.VMEM((SRCN,), jnp.int32),
        pltpu.VMEM((NPIECE, PIECE), jnp.int32),
        pltpu.VMEM((NPIECE, PIECE), jnp.int32),
        pltpu.VMEM((PIECE,), jnp.int32),
        pltpu.VMEM((PIECE,), jnp.float32),
        pltpu.VMEM((PIECE, D), jnp.float32),
        pltpu.VMEM((PIECE, CP), jnp.float32),
        pltpu.SemaphoreType.DMA,
        pltpu.SemaphoreType.DMA,
    ],
)


@jax.jit
def kernel(bx, by_buf, ents_buf, logits_buf, x, y, ents, logits, idx):
    xlgp = jnp.pad(logits, ((0, 0), (0, CP - C)))
    rbx = jax.new_ref(bx)
    rby = jax.new_ref(by_buf)
    ren = jax.new_ref(ents_buf)
    rlg = jax.new_ref(jnp.pad(logits_buf, ((0, 0), (0, CP - C))))
    _sc_update(x, y, ents, xlgp, idx, rbx, rby, ren, rlg)
    return rbx[...], rby[...], ren[...], rlg[...][:, :C]


# speedup vs baseline: 1.0281x; 1.0281x over previous
import jax
import jax.numpy as jnp
from jax import lax
from jax.experimental import pallas as pl
from jax.experimental.pallas import tpu as pltpu
from jax.experimental.pallas import tpu_sc as plsc

M = 100000
D = 256
B = 16384
C = 100
CP = 128

NW = 32
RNG = 3128
CAP = 1024
PIECE = 128
NPIECE = CAP // PIECE
SRCN = 3136


def _sc_body(x_hbm, y_hbm, e_hbm, lg_hbm, idx_hbm,
             rbx, rby, ren, rlg,
             idx_v, src_v, rowl, jl, yv, ev, xbuf, lbuf, semg, sems):
    w = lax.axis_index("s") * 2 + lax.axis_index("c")
    base = jnp.where(w == NW - 1, M - RNG, w * RNG).astype(jnp.int32)
    lane = lax.iota(jnp.int32, 16)

    pltpu.sync_copy(idx_hbm, idx_v)

    def init_body(g, _):
        src_v[pl.ds(g * 16, 16)] = jnp.full((16,), -1, jnp.int32)
        return 0
    lax.fori_loop(0, SRCN // 16, init_body, 0)

    def scan_body(g, _):
        iv = idx_v[pl.ds(g * 16, 16)]
        loc = iv - base
        inr = (loc >= 0) & (loc < RNG)
        j = g * 16 + lane
        _, lastm = plsc.scan_count(loc, mask=inr)
        plsc.store_scatter(src_v, [loc], j, mask=lastm & inr)
        return 0
    lax.fori_loop(0, B // 16, scan_body, 0)

    def comp_body(g, carry):
        k, pmax = carry
        s = src_v[pl.ds(g * 16, 16)]
        m = s >= 0
        ones = jnp.where(m, 1, 0).astype(jnp.int32)
        pos = k + plsc.cumsum(ones) - 1
        m = m & (pos < CAP)
        rows = base + g * 16 + lane
        plsc.store_scatter(rowl, [pos >> 7, pos & 127], rows, mask=m)
        plsc.store_scatter(jl, [pos >> 7, pos & 127], s, mask=m)
        pair = jnp.where(m, rows * 16384 + s, -1)
        return k + jnp.sum(ones), jnp.maximum(pmax, jnp.max(pair))
    k, pmax = lax.fori_loop(0, SRCN // 16, comp_body,
                            (jnp.int32(0), jnp.int32(-1)))

    @pl.when(k > 0)
    def _():
        pad_row = pmax >> 14
        pad_j = pmax & 16383

        def pad_body(g, _):
            posv = g * 16 + lane
            m = posv >= k
            plsc.store_scatter(rowl, [posv >> 7, posv & 127],
                               jnp.full((16,), 1, jnp.int32) * pad_row, mask=m)
            plsc.store_scatter(jl, [posv >> 7, posv & 127],
                               jnp.full((16,), 1, jnp.int32) * pad_j, mask=m)
            return 0
        lax.fori_loop(k >> 4, CAP // 16, pad_body, 0)

        def upd_body(p, _):
            pltpu.make_async_copy(x_hbm.at[jl.at[p]], xbuf, semg).start()
            pltpu.make_async_copy(lg_hbm.at[jl.at[p]], lbuf, semg).start()
            pltpu.make_async_copy(y_hbm.at[jl.at[p]], yv, semg).start()
            pltpu.make_async_copy(e_hbm.at[jl.at[p]], ev, semg).start()
            pltpu.make_async_copy(x_hbm.at[jl.at[p]], xbuf, semg).wait()
            pltpu.make_async_copy(lg_hbm.at[jl.at[p]], lbuf, semg).wait()
            pltpu.make_async_copy(y_hbm.at[jl.at[p]], yv, semg).wait()
            pltpu.make_async_copy(e_hbm.at[jl.at[p]], ev, semg).wait()
            pltpu.make_async_copy(xbuf, rbx.at[rowl.at[p]], sems).start()
            pltpu.make_async_copy(lbuf, rlg.at[rowl.at[p]], sems).start()
            pltpu.make_async_copy(yv, rby.at[rowl.at[p]], sems).start()
            pltpu.make_async_copy(ev, ren.at[rowl.at[p]], sems).start()
            pltpu.make_async_copy(xbuf, rbx.at[rowl.at[p]], sems).wait()
            pltpu.make_async_copy(lbuf, rlg.at[rowl.at[p]], sems).wait()
            pltpu.make_async_copy(yv, rby.at[rowl.at[p]], sems).wait()
            pltpu.make_async_copy(ev, ren.at[rowl.at[p]], sems).wait()
            return 0
        lax.fori_loop(0, (k + PIECE - 1) >> 7, upd_body, 0)


_sc_update = pl.kernel(
    _sc_body,
    out_type=(),
    mesh=plsc.VectorSubcoreMesh(core_axis_name="c", subcore_axis_name="s"),
    compiler_params=pltpu.CompilerParams(needs_layout_passes=False,
                                         use_tc_tiling_on_sc=False),
    scratch_types=[
        pltpu.VMEM((B,), jnp.int32),
        pltpu.VMEM((SRCN,), jnp.int32),
        pltpu.VMEM((NPIECE, PIECE), jnp.int32),
        pltpu.VMEM((NPIECE, PIECE), jnp.int32),
        pltpu.VMEM((PIECE,), jnp.int32),
        pltpu.VMEM((PIECE,), jnp.float32),
        pltpu.VMEM((PIECE, D), jnp.float32),
        pltpu.VMEM((PIECE, CP), jnp.float32),
        pltpu.SemaphoreType.DMA,
        pltpu.SemaphoreType.DMA,
    ],
)


@jax.jit
def kernel(bx, by_buf, ents_buf, logits_buf, x, y, ents, logits, idx):
    xlgp = jnp.pad(logits, ((0, 0), (0, CP - C)))
    rbx = jax.new_ref(bx)
    rby = jax.new_ref(by_buf)
    ren = jax.new_ref(ents_buf)
    rlg = jax.new_ref(jnp.pad(logits_buf, ((0, 0), (0, CP - C))))
    _sc_update(x, y, ents, xlgp, idx, rbx, rby, ren, rlg)
    return rbx[...], rby[...], ren[...], rlg[...][:, :C]
